# baseline (speedup 1.0000x reference)
import functools

import jax
import jax.numpy as jnp
from jax import lax
from jax.experimental import pallas as pl
from jax.experimental.pallas import tpu as pltpu
from jax.experimental.pallas import tpu_sc as plsc

_TOPK = 2
_M = 512
_FN = 1024
_NW = 32


def _ffn_body(gid_ref, valid_ref, bs_ref, ms_ref, ts_ref, x_ref, w1_ref,
              b1_ref, w2_ref, b2_ref, o_ref, xa_ref, xb_ref):
    r = pl.program_id(0)
    ff = pl.program_id(1)

    def gather_block(rr, dst_ref):
        b0 = bs_ref[rr]
        m0 = ms_ref[rr]

        def gath(i, carry):
            tok = ts_ref[jnp.minimum(b0 + i, m0)]
            dst_ref[pl.ds(i, 1), :] = x_ref[pl.ds(tok, 1), :]
            return carry

        lax.fori_loop(0, _M, gath, 0, unroll=8)

    @pl.when(valid_ref[r] != 0)
    def _():
        @pl.when((ff == 0) & (r == 0))
        def _():
            gather_block(0, xa_ref)

        @pl.when(ff == 1)
        def _():
            @pl.when(lax.rem(r + 1, 2) == 0)
            def _():
                gather_block(r + 1, xa_ref)

            @pl.when(lax.rem(r + 1, 2) == 1)
            def _():
                gather_block(r + 1, xb_ref)

        def compute(src_ref):
            h = jnp.dot(src_ref[...], w1_ref[0],
                        preferred_element_type=jnp.float32)
            h = jnp.maximum(h + b1_ref[0, 0, :], 0.0)
            acc = jnp.dot(h, w2_ref[0], preferred_element_type=jnp.float32)

            @pl.when(ff == 0)
            def _():
                o_ref[...] = acc + b2_ref[0, 0, :]

            @pl.when(ff != 0)
            def _():
                o_ref[...] = o_ref[...] + acc

        @pl.when(lax.rem(r, 2) == 0)
        def _():
            compute(xa_ref)

        @pl.when(lax.rem(r, 2) == 1)
        def _():
            compute(xb_ref)


def _grouped_ffn(gid, valid, base_src, max_src, tok_sorted, xf, W1, b1, W2,
                 b2, nb, nff, t, hd, ffd):
    grid_spec = pltpu.PrefetchScalarGridSpec(
        num_scalar_prefetch=5,
        grid=(nb, nff),
        in_specs=[
            pl.BlockSpec((t, hd), lambda r, ff, *pf: (0, 0)),
            pl.BlockSpec((1, hd, _FN), lambda r, ff, g, *pf: (g[r], 0, ff)),
            pl.BlockSpec((1, 1, _FN), lambda r, ff, g, *pf: (g[r], 0, ff)),
            pl.BlockSpec((1, _FN, hd), lambda r, ff, g, *pf: (g[r], ff, 0)),
            pl.BlockSpec((1, 1, hd), lambda r, ff, g, *pf: (g[r], 0, 0)),
        ],
        out_specs=pl.BlockSpec((_M, hd), lambda r, ff, *pf: (r, 0)),
        scratch_shapes=[pltpu.VMEM((_M, hd), jnp.float32),
                        pltpu.VMEM((_M, hd), jnp.float32)],
    )
    return pl.pallas_call(
        _ffn_body,
        grid_spec=grid_spec,
        out_shape=jax.ShapeDtypeStruct((nb * _M, hd), jnp.float32),
        compiler_params=pltpu.CompilerParams(
            dimension_semantics=("arbitrary", "arbitrary")),
    )(gid, valid, base_src, max_src, tok_sorted, xf, W1, b1, W2, b2)


def _make_gather(rows_total, hd):
    rows_per_w = rows_total // _NW
    ns = 6
    rps = 8
    per_round = ns * rps
    rounds = rows_per_w // per_round
    assert rounds * per_round == rows_per_w
    mesh = plsc.VectorSubcoreMesh(core_axis_name="c", subcore_axis_name="s")

    scratch = [pltpu.VMEM((rows_per_w,), jnp.int32)]
    scratch += [pltpu.VMEM((rps, hd), jnp.float32) for _ in range(2 * ns)]
    scratch += [pltpu.SemaphoreType.DMA for _ in range(4 * ns)]

    @functools.partial(
        pl.kernel,
        mesh=mesh,
        out_type=jax.ShapeDtypeStruct((rows_total, hd), jnp.float32),
        scratch_types=scratch,
    )
    def gather_k(x_hbm, idx_hbm, out_hbm, idx_v, *rest):
        bufs = rest[:2 * ns]
        gsems = rest[2 * ns:4 * ns]
        ssems = rest[4 * ns:6 * ns]
        wid = lax.axis_index("s") * 2 + lax.axis_index("c")
        base = wid * rows_per_w
        pltpu.sync_copy(idx_hbm.at[pl.ds(base, rows_per_w)], idx_v)
        g = {}
        st = {}
        for r in range(rounds + 1):
            b = r & 1
            if r < rounds:
                for s in range(ns):
                    if r >= 2:
                        st[(r - 2, s)].wait()
                    off = r * per_round + s * rps
                    g[(r, s)] = pltpu.async_copy(
                        x_hbm.at[idx_v.at[pl.ds(off, rps)]],
                        bufs[b * ns + s], gsems[b * ns + s])
            if r >= 1:
                bp = (r - 1) & 1
                for s in range(ns):
                    g[(r - 1, s)].wait()
                    off = (r - 1) * per_round + s * rps
                    st[(r - 1, s)] = pltpu.async_copy(
                        bufs[bp * ns + s],
                        out_hbm.at[pl.ds(base + off, rps)],
                        ssems[bp * ns + s])
        for s in range(ns):
            st[(rounds - 2, s)].wait()
            st[(rounds - 1, s)].wait()

    return gather_k


def _make_combine(t_total, hd):
    tok_per_w = t_total // _NW
    ch = 16
    n_ch = tok_per_w // ch
    grp = hd // 16
    mesh = plsc.VectorSubcoreMesh(core_axis_name="c", subcore_axis_name="s")

    @functools.partial(
        pl.kernel,
        mesh=mesh,
        out_type=jax.ShapeDtypeStruct((t_total, hd), jnp.float32),
        scratch_types=[
            pltpu.VMEM((tok_per_w,), jnp.int32),
            pltpu.VMEM((tok_per_w,), jnp.int32),
            pltpu.VMEM((ch, hd), jnp.float32),
            pltpu.VMEM((ch, hd), jnp.float32),
            pltpu.VMEM((ch, hd), jnp.float32),
            pltpu.VMEM((ch, hd), jnp.float32),
            pltpu.SemaphoreType.DMA,
            pltpu.SemaphoreType.DMA,
            pltpu.SemaphoreType.DMA,
            pltpu.SemaphoreType.DMA,
            pltpu.SemaphoreType.DMA,
            pltpu.SemaphoreType.DMA,
        ],
    )
    def combine_k(rows_hbm, p0_hbm, p1_hbm, out_hbm, i0_v, i1_v,
                  a0, a1, b0, b1, sga0, sga1, sgb0, sgb1, ss0, ss1):
        wid = lax.axis_index("s") * 2 + lax.axis_index("c")
        base = wid * tok_per_w
        pltpu.sync_copy(p0_hbm.at[pl.ds(base, tok_per_w)], i0_v)
        pltpu.sync_copy(p1_hbm.at[pl.ds(base, tok_per_w)], i1_v)
        abufs, bbufs = (a0, a1), (b0, b1)
        sgas, sgbs, sss = (sga0, sga1), (sgb0, sgb1), (ss0, ss1)
        ga = [None] * n_ch
        gb = [None] * n_ch
        s = [None] * n_ch
        for c in range(n_ch + 1):
            b = c & 1
            if c < n_ch:
                if c >= 2:
                    s[c - 2].wait()
                ga[c] = pltpu.async_copy(
                    rows_hbm.at[i0_v.at[pl.ds(c * ch, ch)]], abufs[b],
                    sgas[b])
                gb[c] = pltpu.async_copy(
                    rows_hbm.at[i1_v.at[pl.ds(c * ch, ch)]], bbufs[b],
                    sgbs[b])
            if c >= 1:
                bp = (c - 1) & 1
                ga[c - 1].wait()
                gb[c - 1].wait()
                av, bv = abufs[bp], bbufs[bp]

                def add_row(i, carry):
                    for j in range(grp):
                        sl = pl.ds(j * 16, 16)
                        av[i, sl] = av[i, sl] + bv[i, sl]
                    return carry

                lax.fori_loop(0, ch, add_row, 0)
                s[c - 1] = pltpu.async_copy(
                    av, out_hbm.at[pl.ds(base + (c - 1) * ch, ch)], sss[bp])
        s[n_ch - 2].wait()
        s[n_ch - 1].wait()

    return combine_k


def kernel(x, gate_W, gate_b, W1, b1, W2, b2):
    bsz, seq, hd = x.shape
    t = bsz * seq
    e = gate_W.shape[1]
    ffd = W1.shape[2]
    na = t * _TOPK
    nb = (na + e * _M) // _M
    rows_total = nb * _M
    nff = ffd // _FN

    xf = x.reshape(t, hd)

    logits = xf @ gate_W + gate_b
    gate_out = jax.nn.softmax(logits, axis=-1)
    _, gate_indices = jax.lax.top_k(gate_out, _TOPK)

    a = gate_indices.reshape(-1).astype(jnp.int32)
    perm = jnp.argsort(a).astype(jnp.int32)
    iperm = jnp.argsort(perm).astype(jnp.int32)
    es = a[perm]
    counts = jnp.bincount(a, length=e).astype(jnp.int32)
    aligned = ((counts + _M - 1) // _M) * _M
    zero1 = jnp.zeros((1,), jnp.int32)
    starts = jnp.concatenate([zero1, jnp.cumsum(aligned)[:-1].astype(jnp.int32)])
    un_starts = jnp.concatenate([zero1, jnp.cumsum(counts)[:-1].astype(jnp.int32)])
    dest = starts[es] + (jnp.arange(na, dtype=jnp.int32) - un_starts[es])
    pos = dest[iperm].reshape(t, _TOPK)
    tok_sorted = (perm // _TOPK).astype(jnp.int32)
    total = jnp.sum(aligned).astype(jnp.int32)
    bm = jnp.arange(nb, dtype=jnp.int32) * _M
    gid_raw = (jnp.sum(bm[:, None] >= starts[None, :], axis=1)
               .astype(jnp.int32) - 1)
    valid = (bm < total).astype(jnp.int32)
    last_gid = gid_raw[jnp.maximum(total // _M - 1, 0)]
    gid = jnp.where(valid != 0, gid_raw, last_gid)
    base_src = (un_starts[gid] - starts[gid] + bm).astype(jnp.int32)
    max_src = jnp.maximum(un_starts[gid] + counts[gid] - 1, 0).astype(jnp.int32)
    base_src = jnp.concatenate([base_src, base_src[-1:]])
    max_src = jnp.concatenate([max_src, max_src[-1:]])

    rows = _grouped_ffn(gid, valid, base_src, max_src, tok_sorted, xf,
                        W1, b1.reshape(e, 1, ffd), W2, b2.reshape(e, 1, hd),
                        nb, nff, t, hd, ffd)
    out = _make_combine(t, hd)(rows, pos[:, 0], pos[:, 1])
    return out.reshape(bsz, seq, hd)


# speedup vs baseline: 1.0706x; 1.0706x over previous
import functools

import jax
import jax.numpy as jnp
from jax import lax
from jax.experimental import pallas as pl
from jax.experimental.pallas import tpu as pltpu
from jax.experimental.pallas import tpu_sc as plsc

_TOPK = 2
_M = 512
_FN = 2048
_NW = 32


def _ffn_body(gid_ref, valid_ref, bs_ref, ms_ref, ts_ref, x_ref, w1_ref,
              b1_ref, w2_ref, b2_ref, o_ref, xblk_ref):
    r = pl.program_id(0)
    ff = pl.program_id(1)

    @pl.when(valid_ref[r] != 0)
    def _():
        @pl.when(ff == 0)
        def _():
            b0 = bs_ref[r]
            m0 = ms_ref[r]

            def gath(i, carry):
                tok = ts_ref[jnp.minimum(b0 + i, m0)]
                xblk_ref[pl.ds(i, 1), :] = x_ref[pl.ds(tok, 1), :]
                return carry

            lax.fori_loop(0, _M, gath, 0, unroll=8)

        h = jnp.dot(xblk_ref[...], w1_ref[0],
                    preferred_element_type=jnp.float32)
        h = jnp.maximum(h + b1_ref[0, 0, :], 0.0)
        acc = jnp.dot(h, w2_ref[0], preferred_element_type=jnp.float32)

        @pl.when(ff == 0)
        def _():
            o_ref[...] = acc + b2_ref[0, 0, :]

        @pl.when(ff != 0)
        def _():
            o_ref[...] = o_ref[...] + acc


def _grouped_ffn(gid, valid, base_src, max_src, tok_sorted, xf, W1, b1, W2,
                 b2, nb, nff, t, hd, ffd):
    grid_spec = pltpu.PrefetchScalarGridSpec(
        num_scalar_prefetch=5,
        grid=(nb, nff),
        in_specs=[
            pl.BlockSpec((t, hd), lambda r, ff, *pf: (0, 0)),
            pl.BlockSpec((1, hd, _FN), lambda r, ff, g, *pf: (g[r], 0, ff)),
            pl.BlockSpec((1, 1, _FN), lambda r, ff, g, *pf: (g[r], 0, ff)),
            pl.BlockSpec((1, _FN, hd), lambda r, ff, g, *pf: (g[r], ff, 0)),
            pl.BlockSpec((1, 1, hd), lambda r, ff, g, *pf: (g[r], 0, 0)),
        ],
        out_specs=pl.BlockSpec((_M, hd), lambda r, ff, *pf: (r, 0)),
        scratch_shapes=[pltpu.VMEM((_M, hd), jnp.float32)],
    )
    return pl.pallas_call(
        _ffn_body,
        grid_spec=grid_spec,
        out_shape=jax.ShapeDtypeStruct((nb * _M, hd), jnp.float32),
        compiler_params=pltpu.CompilerParams(
            dimension_semantics=("arbitrary", "arbitrary")),
    )(gid, valid, base_src, max_src, tok_sorted, xf, W1, b1, W2, b2)


def _make_gather(rows_total, hd):
    rows_per_w = rows_total // _NW
    ns = 6
    rps = 8
    per_round = ns * rps
    rounds = rows_per_w // per_round
    assert rounds * per_round == rows_per_w
    mesh = plsc.VectorSubcoreMesh(core_axis_name="c", subcore_axis_name="s")

    scratch = [pltpu.VMEM((rows_per_w,), jnp.int32)]
    scratch += [pltpu.VMEM((rps, hd), jnp.float32) for _ in range(2 * ns)]
    scratch += [pltpu.SemaphoreType.DMA for _ in range(4 * ns)]

    @functools.partial(
        pl.kernel,
        mesh=mesh,
        out_type=jax.ShapeDtypeStruct((rows_total, hd), jnp.float32),
        scratch_types=scratch,
    )
    def gather_k(x_hbm, idx_hbm, out_hbm, idx_v, *rest):
        bufs = rest[:2 * ns]
        gsems = rest[2 * ns:4 * ns]
        ssems = rest[4 * ns:6 * ns]
        wid = lax.axis_index("s") * 2 + lax.axis_index("c")
        base = wid * rows_per_w
        pltpu.sync_copy(idx_hbm.at[pl.ds(base, rows_per_w)], idx_v)
        g = {}
        st = {}
        for r in range(rounds + 1):
            b = r & 1
            if r < rounds:
                for s in range(ns):
                    if r >= 2:
                        st[(r - 2, s)].wait()
                    off = r * per_round + s * rps
                    g[(r, s)] = pltpu.async_copy(
                        x_hbm.at[idx_v.at[pl.ds(off, rps)]],
                        bufs[b * ns + s], gsems[b * ns + s])
            if r >= 1:
                bp = (r - 1) & 1
                for s in range(ns):
                    g[(r - 1, s)].wait()
                    off = (r - 1) * per_round + s * rps
                    st[(r - 1, s)] = pltpu.async_copy(
                        bufs[bp * ns + s],
                        out_hbm.at[pl.ds(base + off, rps)],
                        ssems[bp * ns + s])
        for s in range(ns):
            st[(rounds - 2, s)].wait()
            st[(rounds - 1, s)].wait()

    return gather_k


def _make_combine(t_total, hd):
    tok_per_w = t_total // _NW
    ch = 16
    n_ch = tok_per_w // ch
    grp = hd // 16
    mesh = plsc.VectorSubcoreMesh(core_axis_name="c", subcore_axis_name="s")

    @functools.partial(
        pl.kernel,
        mesh=mesh,
        out_type=jax.ShapeDtypeStruct((t_total, hd), jnp.float32),
        scratch_types=[
            pltpu.VMEM((tok_per_w,), jnp.int32),
            pltpu.VMEM((tok_per_w,), jnp.int32),
            pltpu.VMEM((ch, hd), jnp.float32),
            pltpu.VMEM((ch, hd), jnp.float32),
            pltpu.VMEM((ch, hd), jnp.float32),
            pltpu.VMEM((ch, hd), jnp.float32),
            pltpu.SemaphoreType.DMA,
            pltpu.SemaphoreType.DMA,
            pltpu.SemaphoreType.DMA,
            pltpu.SemaphoreType.DMA,
            pltpu.SemaphoreType.DMA,
            pltpu.SemaphoreType.DMA,
        ],
    )
    def combine_k(rows_hbm, p0_hbm, p1_hbm, out_hbm, i0_v, i1_v,
                  a0, a1, b0, b1, sga0, sga1, sgb0, sgb1, ss0, ss1):
        wid = lax.axis_index("s") * 2 + lax.axis_index("c")
        base = wid * tok_per_w
        pltpu.sync_copy(p0_hbm.at[pl.ds(base, tok_per_w)], i0_v)
        pltpu.sync_copy(p1_hbm.at[pl.ds(base, tok_per_w)], i1_v)
        abufs, bbufs = (a0, a1), (b0, b1)
        sgas, sgbs, sss = (sga0, sga1), (sgb0, sgb1), (ss0, ss1)
        ga = [None] * n_ch
        gb = [None] * n_ch
        s = [None] * n_ch
        for c in range(n_ch + 1):
            b = c & 1
            if c < n_ch:
                if c >= 2:
                    s[c - 2].wait()
                ga[c] = pltpu.async_copy(
                    rows_hbm.at[i0_v.at[pl.ds(c * ch, ch)]], abufs[b],
                    sgas[b])
                gb[c] = pltpu.async_copy(
                    rows_hbm.at[i1_v.at[pl.ds(c * ch, ch)]], bbufs[b],
                    sgbs[b])
            if c >= 1:
                bp = (c - 1) & 1
                ga[c - 1].wait()
                gb[c - 1].wait()
                av, bv = abufs[bp], bbufs[bp]

                def add_row(i, carry):
                    for j in range(grp):
                        sl = pl.ds(j * 16, 16)
                        av[i, sl] = av[i, sl] + bv[i, sl]
                    return carry

                lax.fori_loop(0, ch, add_row, 0)
                s[c - 1] = pltpu.async_copy(
                    av, out_hbm.at[pl.ds(base + (c - 1) * ch, ch)], sss[bp])
        s[n_ch - 2].wait()
        s[n_ch - 1].wait()

    return combine_k


def kernel(x, gate_W, gate_b, W1, b1, W2, b2):
    bsz, seq, hd = x.shape
    t = bsz * seq
    e = gate_W.shape[1]
    ffd = W1.shape[2]
    na = t * _TOPK
    nb = (na + e * _M) // _M
    rows_total = nb * _M
    nff = ffd // _FN

    xf = x.reshape(t, hd)

    logits = xf @ gate_W + gate_b
    gate_out = jax.nn.softmax(logits, axis=-1)
    _, gate_indices = jax.lax.top_k(gate_out, _TOPK)

    a = gate_indices.reshape(-1).astype(jnp.int32)
    perm = jnp.argsort(a).astype(jnp.int32)
    iperm = jnp.argsort(perm).astype(jnp.int32)
    es = a[perm]
    counts = jnp.bincount(a, length=e).astype(jnp.int32)
    aligned = ((counts + _M - 1) // _M) * _M
    zero1 = jnp.zeros((1,), jnp.int32)
    starts = jnp.concatenate([zero1, jnp.cumsum(aligned)[:-1].astype(jnp.int32)])
    un_starts = jnp.concatenate([zero1, jnp.cumsum(counts)[:-1].astype(jnp.int32)])
    dest = starts[es] + (jnp.arange(na, dtype=jnp.int32) - un_starts[es])
    pos = dest[iperm].reshape(t, _TOPK)
    tok_sorted = (perm // _TOPK).astype(jnp.int32)
    total = jnp.sum(aligned).astype(jnp.int32)
    bm = jnp.arange(nb, dtype=jnp.int32) * _M
    gid_raw = (jnp.sum(bm[:, None] >= starts[None, :], axis=1)
               .astype(jnp.int32) - 1)
    valid = (bm < total).astype(jnp.int32)
    last_gid = gid_raw[jnp.maximum(total // _M - 1, 0)]
    gid = jnp.where(valid != 0, gid_raw, last_gid)
    base_src = (un_starts[gid] - starts[gid] + bm).astype(jnp.int32)
    max_src = jnp.maximum(un_starts[gid] + counts[gid] - 1, 0).astype(jnp.int32)
    base_src = jnp.concatenate([base_src, base_src[-1:]])
    max_src = jnp.concatenate([max_src, max_src[-1:]])

    rows = _grouped_ffn(gid, valid, base_src, max_src, tok_sorted, xf,
                        W1, b1.reshape(e, 1, ffd), W2, b2.reshape(e, 1, hd),
                        nb, nff, t, hd, ffd)
    out = _make_combine(t, hd)(rows, pos[:, 0], pos[:, 1])
    return out.reshape(bsz, seq, hd)
